# speedup vs baseline: 2.4866x; 2.4866x over previous
import jax
import jax.numpy as jnp
from jax.experimental import pallas as pl


def kernel(concat_embeddings, pos_table):
    B, S, H = concat_embeddings.shape
    BS = 256

    def body(x_ref, p_ref, o_ref):
        o_ref[...] = x_ref[...] + p_ref[...]

    out = pl.pallas_call(
        body,
        grid=(S // BS, B),
        in_specs=[
            pl.BlockSpec((1, BS, H), lambda j, b: (b, j, 0)),
            pl.BlockSpec((BS, H), lambda j, b: (j, 0)),
        ],
        out_specs=pl.BlockSpec((1, BS, H), lambda j, b: (b, j, 0)),
        out_shape=jax.ShapeDtypeStruct((B, S, H), concat_embeddings.dtype),
    )(concat_embeddings, pos_table)
    return out
